# Initial kernel scaffold; baseline (speedup 1.0000x reference)
import jax
import jax.numpy as jnp
from jax.experimental import pallas as pl


def kernel(X, emb_tables, categ_idcs, non_categ_mask):
    raise NotImplementedError("write your pallas kernel here")


# speedup vs baseline: 6.3903x; 6.3903x over previous
import functools

import jax
import jax.numpy as jnp
from jax import lax
from jax.experimental import pallas as pl
from jax.experimental.pallas import tpu as pltpu
from jax.experimental.pallas import tpu_sc as plsc

D = 128
N_CAT = 26
D_CONT = D - N_CAT
VOCAB = 1000
EMB_DIM = 32
BATCH = 4096
OUT_D = D_CONT + N_CAT * EMB_DIM

NUM_CORES = 2
NUM_SUBCORES = 16
NW = NUM_CORES * NUM_SUBCORES
ROWS = BATCH // NW
CHUNK = 64
LANES = 16

CAT_BASE = 96
CAT_OFF = D_CONT - CAT_BASE


def _body(x_hbm, tab_hbm, out_hbm, obuf, gath, xcat, idx2d, sem):
    wid = lax.axis_index("s") * NUM_CORES + lax.axis_index("c")
    lanes = lax.iota(jnp.int32, LANES)

    def chunk_pass(c, carry):
        base = wid * ROWS + c * CHUNK

        pltpu.sync_copy(
            x_hbm.at[pl.ds(base, CHUNK), pl.ds(0, D_CONT + 2)],
            obuf.at[:, pl.ds(0, D_CONT + 2)],
        )
        pltpu.sync_copy(
            x_hbm.at[pl.ds(base, CHUNK), pl.ds(CAT_BASE, 32)], xcat
        )

        def feat(i, cc):
            col = jnp.full((LANES,), i + CAT_OFF, jnp.int32)
            off = i * VOCAB

            def sub(m, c2):
                rows = m * LANES + lanes
                v = plsc.load_gather(xcat, [rows, col])
                idx2d[i, pl.ds(m * LANES, LANES)] = v.astype(jnp.int32) + off
                return c2

            return lax.fori_loop(0, CHUNK // LANES, sub, cc)

        lax.fori_loop(0, N_CAT, feat, 0)

        def fire(g, cc):
            pltpu.make_async_copy(
                tab_hbm.at[idx2d.at[g]], gath.at[g], sem
            ).start()
            return cc

        lax.fori_loop(0, N_CAT, fire, 0)

        def drain(g, cc):
            pltpu.make_async_copy(
                tab_hbm.at[idx2d.at[g]], gath.at[g], sem
            ).wait()
            return cc

        lax.fori_loop(0, N_CAT, drain, 0)

        def place_row(r, cc):
            def place_feat(g, c2):
                dst = D_CONT + g * EMB_DIM
                lo = gath[g, r, pl.ds(0, LANES)]
                hi = gath[g, r, pl.ds(LANES, LANES)]
                obuf[r, pl.ds(dst, LANES)] = lo
                obuf[r, pl.ds(dst + LANES, LANES)] = hi
                return c2

            return lax.fori_loop(0, N_CAT, place_feat, cc)

        lax.fori_loop(0, CHUNK, place_row, 0)

        pltpu.sync_copy(obuf, out_hbm.at[pl.ds(base, CHUNK)])
        return carry

    lax.fori_loop(0, ROWS // CHUNK, chunk_pass, 0)


_sc_call = functools.partial(
    pl.kernel,
    mesh=plsc.VectorSubcoreMesh(core_axis_name="c", subcore_axis_name="s"),
    out_type=jax.ShapeDtypeStruct((BATCH, OUT_D), jnp.float32),
    scratch_types=[
        pltpu.VMEM((CHUNK, OUT_D), jnp.float32),
        pltpu.VMEM((N_CAT, CHUNK, EMB_DIM), jnp.float32),
        pltpu.VMEM((CHUNK, 32), jnp.float32),
        pltpu.VMEM((N_CAT, CHUNK), jnp.int32),
        pltpu.SemaphoreType.DMA,
    ],
    compiler_params=pltpu.CompilerParams(
        use_tc_tiling_on_sc=False, needs_layout_passes=False
    ),
)(_body)


@jax.jit
def kernel(X, emb_tables, categ_idcs, non_categ_mask):
    tab = emb_tables.reshape(N_CAT * VOCAB, EMB_DIM)
    return _sc_call(X, tab)
